# Initial kernel scaffold; baseline (speedup 1.0000x reference)
import jax
import jax.numpy as jnp
from jax.experimental import pallas as pl


def kernel(x, table, W, b):
    raise NotImplementedError("write your pallas kernel here")


# speedup vs baseline: 10.8877x; 10.8877x over previous
import functools

import jax
import jax.numpy as jnp
from jax import lax
from jax.experimental import pallas as pl
from jax.experimental.pallas import tpu as pltpu
from jax.experimental.pallas import tpu_sc as plsc

B, S, L = 1024, 20, 20
VOCAB, D, NCLASS = 100000, 128, 100

NIDX = S * L
CHUNK = 100
CPB = NIDX // CHUNK
NW = 32
BPW = B // NW
NCHUNK_W = BPW * CPB
NVREG = D // 16


def _sc_gather_maxpool(x2, table):
    mesh = plsc.VectorSubcoreMesh(core_axis_name="c", subcore_axis_name="s")

    @functools.partial(
        pl.kernel,
        mesh=mesh,
        out_type=jax.ShapeDtypeStruct((B, D), jnp.float32),
        scratch_types=[
            pltpu.VMEM((NCHUNK_W, CHUNK), jnp.int32),
            pltpu.VMEM((CHUNK, D), jnp.float32),
            pltpu.VMEM((CHUNK, D), jnp.float32),
            pltpu.VMEM((BPW, D), jnp.float32),
            pltpu.SemaphoreType.DMA,
            pltpu.SemaphoreType.DMA,
        ],
    )
    def k(x_hbm, table_hbm, out_hbm, idx_v, rows0, rows1, out_v, sem0, sem1):
        wid = lax.axis_index("s") * 2 + lax.axis_index("c")
        pltpu.sync_copy(x_hbm.at[pl.ds(wid * NCHUNK_W, NCHUNK_W)], idx_v)

        rows = (rows0, rows1)
        sems = (sem0, sem1)

        pltpu.async_copy(table_hbm.at[idx_v.at[0]], rows0, sem0)
        pltpu.async_copy(table_hbm.at[idx_v.at[1]], rows1, sem1)

        def reduce_chunk(rref, acc):
            def body(r, acc):
                return tuple(
                    jnp.maximum(acc[j], rref[r, pl.ds(j * 16, 16)])
                    for j in range(NVREG)
                )
            return lax.fori_loop(0, CHUNK, body, acc)

        def batch_body(b, _):
            acc = tuple(
                jnp.full((16,), -jnp.inf, jnp.float32) for _ in range(NVREG)
            )
            for kk in range(CPB):
                c = b * CPB + kk
                buf = kk % 2
                pltpu.make_async_copy(
                    table_hbm.at[idx_v.at[0]], rows[buf], sems[buf]
                ).wait()
                acc = reduce_chunk(rows[buf], acc)
                @pl.when(c + 2 < NCHUNK_W)
                def _():
                    pltpu.async_copy(
                        table_hbm.at[idx_v.at[c + 2]], rows[buf], sems[buf]
                    )
            for j in range(NVREG):
                out_v[b, pl.ds(j * 16, 16)] = acc[j]
            return 0

        lax.fori_loop(0, BPW, batch_body, 0)
        pltpu.sync_copy(out_v, out_hbm.at[pl.ds(wid * BPW, BPW)])

    return k(x2, table)


def _fc_sigmoid(h, W, b):
    def fc_kernel(h_ref, w_ref, b_ref, o_ref):
        acc = lax.dot_general(
            h_ref[...], w_ref[...],
            dimension_numbers=(((1,), (1,)), ((), ())),
            preferred_element_type=jnp.float32,
        )
        o_ref[...] = jax.nn.sigmoid(acc + b_ref[...])

    return pl.pallas_call(
        fc_kernel,
        out_shape=jax.ShapeDtypeStruct((B, NCLASS), jnp.float32),
    )(h, W, b.reshape(1, NCLASS))


def kernel(x, table, W, b):
    x2 = x.astype(jnp.int32).reshape(B * CPB, CHUNK)
    h = _sc_gather_maxpool(x2, table)
    return _fc_sigmoid(h, W, b)


# speedup vs baseline: 14.8635x; 1.3652x over previous
import functools

import jax
import jax.numpy as jnp
from jax import lax
from jax.experimental import pallas as pl
from jax.experimental.pallas import tpu as pltpu
from jax.experimental.pallas import tpu_sc as plsc

B, S, L = 1024, 20, 20
VOCAB, D, NCLASS = 100000, 128, 100

NIDX = S * L
CHUNK = 100
CPB = NIDX // CHUNK
NW = 32
BPW = B // NW
NCHUNK_W = BPW * CPB
NVREG = D // 16


def _sc_gather_maxpool(x2, table):
    mesh = plsc.VectorSubcoreMesh(core_axis_name="c", subcore_axis_name="s")

    @functools.partial(
        pl.kernel,
        mesh=mesh,
        out_type=jax.ShapeDtypeStruct((B, D), jnp.float32),
        scratch_types=[
            pltpu.VMEM((NCHUNK_W, CHUNK), jnp.int32),
            pltpu.VMEM((CHUNK, D), jnp.float32),
            pltpu.VMEM((CHUNK, D), jnp.float32),
            pltpu.VMEM((CHUNK, D), jnp.float32),
            pltpu.VMEM((CHUNK, D), jnp.float32),
            pltpu.VMEM((BPW, D), jnp.float32),
            pltpu.SemaphoreType.DMA,
            pltpu.SemaphoreType.DMA,
            pltpu.SemaphoreType.DMA,
            pltpu.SemaphoreType.DMA,
        ],
    )
    def k(x_hbm, table_hbm, out_hbm, idx_v, rows0, rows1, rows2, rows3,
          out_v, sem0, sem1, sem2, sem3):
        wid = lax.axis_index("s") * 2 + lax.axis_index("c")
        pltpu.sync_copy(x_hbm.at[pl.ds(wid * NCHUNK_W, NCHUNK_W)], idx_v)

        rows = (rows0, rows1, rows2, rows3)
        sems = (sem0, sem1, sem2, sem3)

        for kk in range(CPB):
            pltpu.async_copy(table_hbm.at[idx_v.at[kk]], rows[kk], sems[kk])

        UNROLL = 4

        def reduce_chunk(rref, acc):
            def body(r, acc):
                for u in range(UNROLL):
                    acc = tuple(
                        jnp.maximum(acc[j], rref[r * UNROLL + u, pl.ds(j * 16, 16)])
                        for j in range(NVREG)
                    )
                return acc
            return lax.fori_loop(0, CHUNK // UNROLL, body, acc)

        def batch_body(b, _):
            acc = tuple(
                jnp.full((16,), -jnp.inf, jnp.float32) for _ in range(NVREG)
            )
            for kk in range(CPB):
                c = b * CPB + kk
                buf = kk
                pltpu.make_async_copy(
                    table_hbm.at[idx_v.at[0]], rows[buf], sems[buf]
                ).wait()
                acc = reduce_chunk(rows[buf], acc)
                @pl.when(c + CPB < NCHUNK_W)
                def _():
                    pltpu.async_copy(
                        table_hbm.at[idx_v.at[c + CPB]], rows[buf], sems[buf]
                    )
            for j in range(NVREG):
                out_v[b, pl.ds(j * 16, 16)] = acc[j]
            return 0

        lax.fori_loop(0, BPW, batch_body, 0)
        pltpu.sync_copy(out_v, out_hbm.at[pl.ds(wid * BPW, BPW)])

    return k(x2, table)


def _fc_sigmoid(h, W, b):
    def fc_kernel(h_ref, w_ref, b_ref, o_ref):
        acc = lax.dot_general(
            h_ref[...], w_ref[...],
            dimension_numbers=(((1,), (1,)), ((), ())),
            preferred_element_type=jnp.float32,
        )
        o_ref[...] = jax.nn.sigmoid(acc + b_ref[...])

    return pl.pallas_call(
        fc_kernel,
        out_shape=jax.ShapeDtypeStruct((B, NCLASS), jnp.float32),
    )(h, W, b.reshape(1, NCLASS))


def kernel(x, table, W, b):
    x2 = x.astype(jnp.int32).reshape(B * CPB, CHUNK)
    h = _sc_gather_maxpool(x2, table)
    return _fc_sigmoid(h, W, b)


# speedup vs baseline: 14.8867x; 1.0016x over previous
import functools

import jax
import jax.numpy as jnp
from jax import lax
from jax.experimental import pallas as pl
from jax.experimental.pallas import tpu as pltpu
from jax.experimental.pallas import tpu_sc as plsc

B, S, L = 1024, 20, 20
VOCAB, D, NCLASS = 100000, 128, 100

NIDX = S * L
CHUNK = 80
CPB = NIDX // CHUNK
NW = 32
BPW = B // NW
IDX_W = BPW * NIDX
NCHUNK_W = BPW * CPB
NVREG = D // 16
UNROLL = 4


def _sc_gather_maxpool(xf, table):
    mesh = plsc.VectorSubcoreMesh(core_axis_name="c", subcore_axis_name="s")

    @functools.partial(
        pl.kernel,
        mesh=mesh,
        out_type=jax.ShapeDtypeStruct((B, D), jnp.float32),
        scratch_types=[
            pltpu.VMEM((IDX_W,), jnp.int32),
            pltpu.VMEM((CHUNK, D), jnp.float32),
            pltpu.VMEM((CHUNK, D), jnp.float32),
            pltpu.VMEM((CHUNK, D), jnp.float32),
            pltpu.VMEM((CHUNK, D), jnp.float32),
            pltpu.VMEM((CHUNK, D), jnp.float32),
            pltpu.VMEM((BPW, D), jnp.float32),
            pltpu.SemaphoreType.DMA,
            pltpu.SemaphoreType.DMA,
            pltpu.SemaphoreType.DMA,
            pltpu.SemaphoreType.DMA,
            pltpu.SemaphoreType.DMA,
        ],
    )
    def k(x_hbm, table_hbm, out_hbm, idx_v, rows0, rows1, rows2, rows3,
          rows4, out_v, sem0, sem1, sem2, sem3, sem4):
        wid = lax.axis_index("s") * 2 + lax.axis_index("c")
        pltpu.sync_copy(x_hbm.at[pl.ds(wid * IDX_W, IDX_W)], idx_v)

        rows = (rows0, rows1, rows2, rows3, rows4)
        sems = (sem0, sem1, sem2, sem3, sem4)

        for kk in range(CPB):
            pltpu.async_copy(
                table_hbm.at[idx_v.at[pl.ds(kk * CHUNK, CHUNK)]],
                rows[kk], sems[kk],
            )

        def reduce_chunk(rref, acc):
            def body(r, acc):
                for u in range(UNROLL):
                    acc = tuple(
                        jnp.maximum(acc[j], rref[r * UNROLL + u, pl.ds(j * 16, 16)])
                        for j in range(NVREG)
                    )
                return acc
            return lax.fori_loop(0, CHUNK // UNROLL, body, acc)

        def batch_body(b, _):
            acc = tuple(
                jnp.full((16,), -jnp.inf, jnp.float32) for _ in range(NVREG)
            )
            for kk in range(CPB):
                c = b * CPB + kk
                buf = kk
                pltpu.make_async_copy(
                    table_hbm.at[idx_v.at[pl.ds(0, CHUNK)]], rows[buf], sems[buf]
                ).wait()
                acc = reduce_chunk(rows[buf], acc)
                @pl.when(c + CPB < NCHUNK_W)
                def _():
                    pltpu.async_copy(
                        table_hbm.at[idx_v.at[pl.ds((c + CPB) * CHUNK, CHUNK)]],
                        rows[buf], sems[buf],
                    )
            for j in range(NVREG):
                out_v[b, pl.ds(j * 16, 16)] = acc[j]
            return 0

        lax.fori_loop(0, BPW, batch_body, 0)
        pltpu.sync_copy(out_v, out_hbm.at[pl.ds(wid * BPW, BPW)])

    return k(xf, table)


def _fc_sigmoid(h, W, b):
    def fc_kernel(h_ref, w_ref, b_ref, o_ref):
        acc = lax.dot_general(
            h_ref[...], w_ref[...],
            dimension_numbers=(((1,), (1,)), ((), ())),
            preferred_element_type=jnp.float32,
        )
        o_ref[...] = jax.nn.sigmoid(acc + b_ref[...])

    return pl.pallas_call(
        fc_kernel,
        out_shape=jax.ShapeDtypeStruct((B, NCLASS), jnp.float32),
    )(h, W, b.reshape(1, NCLASS))


def kernel(x, table, W, b):
    xf = x.astype(jnp.int32).reshape(B * NIDX)
    h = _sc_gather_maxpool(xf, table)
    return _fc_sigmoid(h, W, b)
